# baseline (speedup 1.0000x reference)
import functools

import jax
import jax.numpy as jnp
from jax import lax
from jax.experimental import pallas as pl
from jax.experimental.pallas import tpu as pltpu
from jax.experimental.pallas import tpu_sc as plsc

B = 4096
E = 128
SD = 128
NEG = 5

NC = 2
NS = 16
NW = NC * NS

PH = 2
HB = B // PH
BPW = HB // NW


@functools.cache
def _build_sc_gather():
    mesh = plsc.VectorSubcoreMesh(core_axis_name="c", subcore_axis_name="s")

    @functools.partial(
        pl.kernel,
        mesh=mesh,
        out_type=(
            jax.ShapeDtypeStruct((HB, E), jnp.float32),
            jax.ShapeDtypeStruct((HB, E), jnp.float32),
            jax.ShapeDtypeStruct((HB, SD), jnp.float32),
            jax.ShapeDtypeStruct((NEG * HB, E), jnp.float32),
            jax.ShapeDtypeStruct((NEG * HB, SD), jnp.float32),
        ),
        scratch_types=[
            pltpu.VMEM((7, BPW), jnp.int32),
            pltpu.VMEM((BPW, E), jnp.float32),
            pltpu.VMEM((BPW, E), jnp.float32),
            pltpu.SemaphoreType.DMA,
            pltpu.SemaphoreType.DMA,
        ],
    )
    def _sc_gather(idx_all_h, WI_h, WO_h, SE_h,
                   vI_h, WOy_h, SEy_h, WOn_h, SEn_h,
                   idx_v, buf0, buf1, sem0, sem1):
        wid = lax.axis_index("s") * NC + lax.axis_index("c")
        base = wid * BPW

        pltpu.sync_copy(idx_all_h.at[wid], idx_v)

        tasks = [(0, WI_h, vI_h, base), (1, WO_h, WOy_h, base),
                 (1, SE_h, SEy_h, base)]
        for n in range(NEG):
            tasks.append((2 + n, WO_h, WOn_h, n * HB + base))
        for n in range(NEG):
            tasks.append((2 + n, SE_h, SEn_h, n * HB + base))

        bufs = (buf0, buf1)
        sems = (sem0, sem1)
        copies = [None, None]

        def start(t, slot):
            j, tab_h, _, _ = tasks[t]
            copies[slot] = pltpu.async_copy(tab_h.at[idx_v.at[j]],
                                            bufs[slot], sems[slot])

        start(0, 0)
        for t in range(len(tasks)):
            slot = t % 2
            if t + 1 < len(tasks):
                start(t + 1, 1 - slot)
            copies[slot].wait()
            _, _, out_h, ooff = tasks[t]
            pltpu.sync_copy(bufs[slot], out_h.at[pl.ds(ooff, BPW)])

    return _sc_gather


BC = 512


def _log_sigmoid(z):
    return jnp.minimum(z, 0.0) - jnp.log1p(jnp.exp(-jnp.abs(z)))


def _tc_body(vI_r, WOy_r, SEy_r, WOn_r, SEn_r, f1w_r, f2w_r, f1b_r, f2b_r,
             out_r):
    i = pl.program_id(0)
    vI = vI_r[...]
    U = jnp.dot(vI, f2w_r[...], preferred_element_type=jnp.float32)
    U1 = U[:, :E]
    U2 = U[:, E:]
    f1w = f1w_r[...]
    A1 = jnp.dot(U1, f1w, preferred_element_type=jnp.float32)
    A2 = jnp.dot(U2, f1w, preferred_element_type=jnp.float32)
    f1b = f1b_r[...]
    f2b = f2b_r[...]
    cI = jnp.sum(vI * f2b, axis=1)
    c1 = jnp.sum(U1 * f1b, axis=1)
    c2 = jnp.sum(U2 * f1b, axis=1)
    pos = jnp.sum(U1 * WOy_r[...] + A2 * SEy_r[...], axis=1) + c2 + cI
    part = -jnp.sum(_log_sigmoid(pos)) / B
    cneg = c1 + cI
    for n in range(NEG):
        zn = jnp.sum(A1 * WOn_r[n] + U2 * SEn_r[n], axis=1) + cneg
        part = part - jnp.sum(_log_sigmoid(-zn))

    @pl.when(i == 0)
    def _init():
        out_r[0, 0] = part

    @pl.when(i > 0)
    def _acc():
        out_r[0, 0] = out_r[0, 0] + part


def _tc_compute(vI, WOy, SEy, WOn, SEn, f1w, f2w, f1b, f2b):
    grid = (HB // BC,)
    return pl.pallas_call(
        _tc_body,
        grid=grid,
        in_specs=[
            pl.BlockSpec((BC, E), lambda i: (i, 0)),
            pl.BlockSpec((BC, E), lambda i: (i, 0)),
            pl.BlockSpec((BC, SD), lambda i: (i, 0)),
            pl.BlockSpec((NEG, BC, E), lambda i: (0, i, 0)),
            pl.BlockSpec((NEG, BC, SD), lambda i: (0, i, 0)),
            pl.BlockSpec((SD, SD), lambda i: (0, 0)),
            pl.BlockSpec((E, E + SD), lambda i: (0, 0)),
            pl.BlockSpec((1, SD), lambda i: (0, 0)),
            pl.BlockSpec((1, E), lambda i: (0, 0)),
        ],
        out_specs=pl.BlockSpec((1, 1), lambda i: (0, 0),
                               memory_space=pltpu.SMEM),
        out_shape=jax.ShapeDtypeStruct((1, 1), jnp.float32),
    )(vI, WOy, SEy, WOn, SEn, f1w, f2w, f1b, f2b)


def kernel(x, y, neg, WI, WO, seq_embed, fc1_w, fc1_b, fc2_w, fc2_b):
    xi = x.astype(jnp.int32).reshape(PH, NW, 1, BPW)
    yi = y.astype(jnp.int32).reshape(PH, NW, 1, BPW)
    negr = (neg.astype(jnp.int32).reshape(PH, NW, BPW, NEG)
            .transpose(0, 1, 3, 2))
    idx_all = jnp.concatenate([xi, yi, negr], axis=2)
    f1b = fc1_b.reshape(1, SD)
    f2b = fc2_b.reshape(1, E)
    sc = _build_sc_gather()
    total = None
    for h in range(PH):
        vI, WOy, SEy, WOn, SEn = sc(idx_all[h], WI, WO, seq_embed)
        part = _tc_compute(vI, WOy, SEy,
                           WOn.reshape(NEG, HB, E), SEn.reshape(NEG, HB, SD),
                           fc1_w, fc2_w, f1b, f2b)[0, 0]
        total = part if total is None else total + part
    return total


# speedup vs baseline: 1.1264x; 1.1264x over previous
import functools

import jax
import jax.numpy as jnp
from jax import lax
from jax.experimental import pallas as pl
from jax.experimental.pallas import tpu as pltpu
from jax.experimental.pallas import tpu_sc as plsc

B = 4096
E = 128
SD = 128
NEG = 5

NC = 2
NS = 16
NW = NC * NS

PH = 1
HB = B // PH
BPW = HB // NW


@functools.cache
def _build_sc_gather():
    mesh = plsc.VectorSubcoreMesh(core_axis_name="c", subcore_axis_name="s")

    @functools.partial(
        pl.kernel,
        mesh=mesh,
        out_type=(
            jax.ShapeDtypeStruct((HB, E), jnp.float32),
            jax.ShapeDtypeStruct((HB, E), jnp.float32),
            jax.ShapeDtypeStruct((HB, SD), jnp.float32),
            jax.ShapeDtypeStruct((NEG * HB, E), jnp.float32),
            jax.ShapeDtypeStruct((NEG * HB, SD), jnp.float32),
        ),
        scratch_types=[
            pltpu.VMEM((7, BPW), jnp.int32),
            pltpu.VMEM((BPW, E), jnp.float32),
            pltpu.VMEM((BPW, E), jnp.float32),
            pltpu.SemaphoreType.DMA,
            pltpu.SemaphoreType.DMA,
        ],
    )
    def _sc_gather(idx_all_h, WI_h, WO_h, SE_h,
                   vI_h, WOy_h, SEy_h, WOn_h, SEn_h,
                   idx_v, buf0, buf1, sem0, sem1):
        wid = lax.axis_index("s") * NC + lax.axis_index("c")
        base = wid * BPW

        pltpu.sync_copy(idx_all_h.at[wid], idx_v)

        tasks = [(0, WI_h, vI_h, base), (1, WO_h, WOy_h, base),
                 (1, SE_h, SEy_h, base)]
        for n in range(NEG):
            tasks.append((2 + n, WO_h, WOn_h, n * HB + base))
        for n in range(NEG):
            tasks.append((2 + n, SE_h, SEn_h, n * HB + base))

        bufs = (buf0, buf1)
        sems = (sem0, sem1)
        copies = [None, None]

        def start(t, slot):
            j, tab_h, _, _ = tasks[t]
            copies[slot] = pltpu.async_copy(tab_h.at[idx_v.at[j]],
                                            bufs[slot], sems[slot])

        start(0, 0)
        for t in range(len(tasks)):
            slot = t % 2
            if t + 1 < len(tasks):
                start(t + 1, 1 - slot)
            copies[slot].wait()
            _, _, out_h, ooff = tasks[t]
            pltpu.sync_copy(bufs[slot], out_h.at[pl.ds(ooff, BPW)])

    return _sc_gather


BC = 512


def _log_sigmoid(z):
    return jnp.minimum(z, 0.0) - jnp.log1p(jnp.exp(-jnp.abs(z)))


def _tc_body(vI_r, WOy_r, SEy_r, WOn_r, SEn_r, f1w_r, f2w_r, f1b_r, f2b_r,
             out_r):
    i = pl.program_id(0)
    vI = vI_r[...]
    U = jnp.dot(vI, f2w_r[...], preferred_element_type=jnp.float32)
    U1 = U[:, :E]
    U2 = U[:, E:]
    f1w = f1w_r[...]
    A1 = jnp.dot(U1, f1w, preferred_element_type=jnp.float32)
    A2 = jnp.dot(U2, f1w, preferred_element_type=jnp.float32)
    f1b = f1b_r[...]
    f2b = f2b_r[...]
    cI = jnp.sum(vI * f2b, axis=1)
    c1 = jnp.sum(U1 * f1b, axis=1)
    c2 = jnp.sum(U2 * f1b, axis=1)
    pos = jnp.sum(U1 * WOy_r[...] + A2 * SEy_r[...], axis=1) + c2 + cI
    part = -jnp.sum(_log_sigmoid(pos)) / B
    cneg = c1 + cI
    for n in range(NEG):
        zn = jnp.sum(A1 * WOn_r[n] + U2 * SEn_r[n], axis=1) + cneg
        part = part - jnp.sum(_log_sigmoid(-zn))

    @pl.when(i == 0)
    def _init():
        out_r[0, 0] = part

    @pl.when(i > 0)
    def _acc():
        out_r[0, 0] = out_r[0, 0] + part


def _tc_compute(vI, WOy, SEy, WOn, SEn, f1w, f2w, f1b, f2b):
    grid = (HB // BC,)
    return pl.pallas_call(
        _tc_body,
        grid=grid,
        in_specs=[
            pl.BlockSpec((BC, E), lambda i: (i, 0)),
            pl.BlockSpec((BC, E), lambda i: (i, 0)),
            pl.BlockSpec((BC, SD), lambda i: (i, 0)),
            pl.BlockSpec((NEG, BC, E), lambda i: (0, i, 0)),
            pl.BlockSpec((NEG, BC, SD), lambda i: (0, i, 0)),
            pl.BlockSpec((SD, SD), lambda i: (0, 0)),
            pl.BlockSpec((E, E + SD), lambda i: (0, 0)),
            pl.BlockSpec((1, SD), lambda i: (0, 0)),
            pl.BlockSpec((1, E), lambda i: (0, 0)),
        ],
        out_specs=pl.BlockSpec((1, 1), lambda i: (0, 0),
                               memory_space=pltpu.SMEM),
        out_shape=jax.ShapeDtypeStruct((1, 1), jnp.float32),
    )(vI, WOy, SEy, WOn, SEn, f1w, f2w, f1b, f2b)


def kernel(x, y, neg, WI, WO, seq_embed, fc1_w, fc1_b, fc2_w, fc2_b):
    xi = x.astype(jnp.int32).reshape(PH, NW, 1, BPW)
    yi = y.astype(jnp.int32).reshape(PH, NW, 1, BPW)
    negr = (neg.astype(jnp.int32).reshape(PH, NW, BPW, NEG)
            .transpose(0, 1, 3, 2))
    idx_all = jnp.concatenate([xi, yi, negr], axis=2)
    f1b = fc1_b.reshape(1, SD)
    f2b = fc2_b.reshape(1, E)
    sc = _build_sc_gather()
    total = None
    for h in range(PH):
        vI, WOy, SEy, WOn, SEn = sc(idx_all[h], WI, WO, seq_embed)
        part = _tc_compute(vI, WOy, SEy,
                           WOn.reshape(NEG, HB, E), SEn.reshape(NEG, HB, SD),
                           fc1_w, fc2_w, f1b, f2b)[0, 0]
        total = part if total is None else total + part
    return total
